# baseline (speedup 1.0000x reference)
import jax, jax.numpy as jnp
import numpy as np
from jax.experimental import pallas as pl

N = 10000
NUM_GRAPHS = 64

def _bn(x, g, b):
    m = jnp.mean(x, axis=0)
    v = jnp.var(x, axis=0)
    return (x - m) / jnp.sqrt(v + 1e-5) * g + b

def _gin_conv(x, src, dst, W, b):
    agg = jax.ops.segment_sum(x[src], dst, num_segments=x.shape[0])
    return (x + agg) @ W + b

def kernel(x, edge_index, batch, W1, b1, g1, be1, W2, b2, g2, be2, W3, b3, g3, be3, Wf2, bf2, g4, be4, Wf3, bf3):
    src, dst = edge_index[0], edge_index[1]
    h = jax.nn.relu(_bn(_gin_conv(x, src, dst, W1, b1), g1, be1))
    h = jax.nn.relu(_bn(_gin_conv(h, src, dst, W2, b2), g2, be2))
    h = jax.nn.relu(_bn(_gin_conv(h, src, dst, W3, b3), g3, be3))
    hg = jax.ops.segment_sum(h, batch, num_segments=NUM_GRAPHS)
    hg = jax.nn.softplus(hg)
    hg = jax.nn.softplus(_bn(hg @ Wf2 + bf2, g4, be4))
    hg = hg @ Wf3 + bf3
    norm = jnp.maximum(jnp.linalg.norm(hg, axis=1, keepdims=True), 1e-12)
    return hg / norm


# speedup vs baseline: 3.1499x; 3.1499x over previous
import functools

import jax
import jax.numpy as jnp
from jax import lax
from jax.experimental import pallas as pl
from jax.experimental.pallas import tpu as pltpu
from jax.experimental.pallas import tpu_sc as plsc

N = 10000
E = 320000
NUM_GRAPHS = 64

NC = 2
NS = 16
K = 128
EPAD = 4096 * 79
NPAD = 10240
RPT = NPAD // NS
DH = 128

_MESH = plsc.VectorSubcoreMesh(core_axis_name="c", subcore_axis_name="s")


def _make_agg(split_cores):
    TPT = EPAD // (NC * NS) if split_cores else EPAD // NS
    TRIPS = TPT // K

    @functools.partial(
        pl.kernel,
        out_type=jax.ShapeDtypeStruct((NC, NPAD, DH), jnp.float32),
        mesh=_MESH,
        scratch_types=[
            pltpu.VMEM((K,), jnp.int32),
            pltpu.VMEM((K,), jnp.int32),
            pltpu.VMEM((K, DH), jnp.float32),
            pltpu.VMEM_SHARED((NPAD, DH), jnp.float32),
            pltpu.SemaphoreType.DMA,
        ],
    )
    def agg(h_hbm, src_hbm, dst_hbm, zero_hbm, out_hbm,
            idx_v, dst_v, stage, acc_sh, sem):
        cidx = lax.axis_index("c")
        sidx = lax.axis_index("s")

        rbase = sidx * RPT
        pltpu.sync_copy(zero_hbm.at[pl.ds(rbase, RPT)],
                        acc_sh.at[pl.ds(rbase, RPT)])
        plsc.subcore_barrier()

        if split_cores:
            ebase = (cidx * NS + sidx) * TPT
        else:
            ebase = sidx * TPT

        def body(g, carry):
            e0 = ebase + g * K
            pltpu.sync_copy(src_hbm.at[cidx, pl.ds(e0, K)], idx_v)
            pltpu.sync_copy(dst_hbm.at[pl.ds(e0, K)], dst_v)
            pltpu.async_copy(h_hbm.at[idx_v], stage, sem).wait()
            pltpu.sync_copy(stage, acc_sh.at[dst_v], add=True)
            return carry

        lax.fori_loop(0, TRIPS, body, 0)
        plsc.subcore_barrier()

        pltpu.sync_copy(acc_sh.at[pl.ds(rbase, RPT)],
                        out_hbm.at[cidx, pl.ds(rbase, RPT)])

    return agg


_agg_split = _make_agg(True)
_agg_cols = _make_agg(False)


R = 400
GRID = N // R


def _acc_stats(y, s_ref):
    @pl.when(pl.program_id(0) == 0)
    def _():
        s_ref[...] = jnp.zeros_like(s_ref)

    s_ref[0:1, :] += jnp.sum(y, axis=0, keepdims=True)
    s_ref[1:2, :] += jnp.sum(y * y, axis=0, keepdims=True)


_PREC = lax.Precision.HIGHEST


def _mm1_body(x_ref, a_ref, w_ref, b_ref, y_ref, s_ref):
    s = x_ref[...] + a_ref[0] + a_ref[1]
    y = jnp.dot(s, w_ref[...], preferred_element_type=jnp.float32) + b_ref[...]
    y_ref[...] = y
    _acc_stats(y, s_ref)


def _mmstats1(x, agg, W, b):
    Din, Dout = W.shape
    return pl.pallas_call(
        _mm1_body,
        grid=(GRID,),
        in_specs=[
            pl.BlockSpec((R, Din), lambda i: (i, 0)),
            pl.BlockSpec((NC, R, Din), lambda i: (0, i, 0)),
            pl.BlockSpec((Din, Dout), lambda i: (0, 0)),
            pl.BlockSpec((1, Dout), lambda i: (0, 0)),
        ],
        out_specs=[
            pl.BlockSpec((R, Dout), lambda i: (i, 0)),
            pl.BlockSpec((2, Dout), lambda i: (0, 0)),
        ],
        out_shape=[
            jax.ShapeDtypeStruct((N, Dout), jnp.float32),
            jax.ShapeDtypeStruct((2, Dout), jnp.float32),
        ],
    )(x, agg, W, b.reshape(1, Dout))


def _mm23_body(h_ref, a_ref, w_ref, b_ref, y_ref, s_ref):
    s0 = h_ref[0] + a_ref[0]
    s1 = h_ref[1] + a_ref[1]
    y = (jnp.dot(s0, w_ref[0:DH, :], preferred_element_type=jnp.float32)
         + jnp.dot(s1, w_ref[DH:2 * DH, :], preferred_element_type=jnp.float32)
         + b_ref[...])
    y_ref[...] = y
    _acc_stats(y, s_ref)


def _mmstats23(h, agg, W, b):
    Dout = W.shape[1]
    return pl.pallas_call(
        _mm23_body,
        grid=(GRID,),
        in_specs=[
            pl.BlockSpec((NC, R, DH), lambda i: (0, i, 0)),
            pl.BlockSpec((NC, R, DH), lambda i: (0, i, 0)),
            pl.BlockSpec((2 * DH, Dout), lambda i: (0, 0)),
            pl.BlockSpec((1, Dout), lambda i: (0, 0)),
        ],
        out_specs=[
            pl.BlockSpec((R, Dout), lambda i: (i, 0)),
            pl.BlockSpec((2, Dout), lambda i: (0, 0)),
        ],
        out_shape=[
            jax.ShapeDtypeStruct((N, Dout), jnp.float32),
            jax.ShapeDtypeStruct((2, Dout), jnp.float32),
        ],
    )(h, agg, W, b.reshape(1, Dout))


def _bn_from_stats(s_ref, y):
    m = s_ref[0:1, :] / N
    v = s_ref[1:2, :] / N - m * m
    r = lax.rsqrt(v + 1e-5)
    return (y - m) * r


def _bnrelu_body(y_ref, s_ref, g_ref, be_ref, h_ref):
    h = _bn_from_stats(s_ref, y_ref[...]) * g_ref[...] + be_ref[...]
    h = jnp.maximum(h, 0.0)
    h_ref[0] = h[:, 0:DH]
    h_ref[1] = h[:, DH:2 * DH]


def _bnrelu(y, s, g, be):
    Dout = y.shape[1]
    return pl.pallas_call(
        _bnrelu_body,
        grid=(GRID,),
        in_specs=[
            pl.BlockSpec((R, Dout), lambda i: (i, 0)),
            pl.BlockSpec((2, Dout), lambda i: (0, 0)),
            pl.BlockSpec((1, Dout), lambda i: (0, 0)),
            pl.BlockSpec((1, Dout), lambda i: (0, 0)),
        ],
        out_specs=pl.BlockSpec((NC, R, DH), lambda i: (0, i, 0)),
        out_shape=jax.ShapeDtypeStruct((NC, N, DH), jnp.float32),
    )(y, s, g.reshape(1, Dout), be.reshape(1, Dout))


def _pool_body(y_ref, s_ref, g_ref, be_ref, b_ref, hg_ref):
    h = _bn_from_stats(s_ref, y_ref[...]) * g_ref[...] + be_ref[...]
    h = jnp.maximum(h, 0.0)
    bb = b_ref[0]
    gids = lax.broadcasted_iota(jnp.int32, (NUM_GRAPHS, R), 0)
    oh = (bb == gids).astype(jnp.float32)

    @pl.when(pl.program_id(0) == 0)
    def _():
        hg_ref[...] = jnp.zeros_like(hg_ref)

    hg_ref[...] += jnp.dot(oh, h, precision=_PREC,
                           preferred_element_type=jnp.float32)


def _pool(y, s, g, be, batch3d):
    Dout = y.shape[1]
    return pl.pallas_call(
        _pool_body,
        grid=(GRID,),
        in_specs=[
            pl.BlockSpec((R, Dout), lambda i: (i, 0)),
            pl.BlockSpec((2, Dout), lambda i: (0, 0)),
            pl.BlockSpec((1, Dout), lambda i: (0, 0)),
            pl.BlockSpec((1, Dout), lambda i: (0, 0)),
            pl.BlockSpec((1, 1, R), lambda i: (i, 0, 0)),
        ],
        out_specs=pl.BlockSpec((NUM_GRAPHS, Dout), lambda i: (0, 0)),
        out_shape=jax.ShapeDtypeStruct((NUM_GRAPHS, Dout), jnp.float32),
    )(y, s, g.reshape(1, Dout), be.reshape(1, Dout), batch3d)


def _softplus(x):
    return jnp.maximum(x, 0.0) + jnp.log1p(jnp.exp(-jnp.abs(x)))


def _head_body(hg_ref, w2_ref, b2_ref, g4_ref, be4_ref, w3_ref, b3_ref, o_ref):
    sp = _softplus(hg_ref[...])
    z = jnp.dot(sp, w2_ref[...], preferred_element_type=jnp.float32) + b2_ref[...]
    m = jnp.mean(z, axis=0, keepdims=True)
    v = jnp.mean(z * z, axis=0, keepdims=True) - m * m
    z = (z - m) * lax.rsqrt(v + 1e-5) * g4_ref[...] + be4_ref[...]
    z = _softplus(z)
    o = jnp.dot(z, w3_ref[...], preferred_element_type=jnp.float32) + b3_ref[...]
    nrm = jnp.maximum(jnp.sqrt(jnp.sum(o * o, axis=1, keepdims=True)), 1e-12)
    o_ref[...] = o / nrm


def _head(hg, Wf2, bf2, g4, be4, Wf3, bf3):
    Dh, Dfc = Wf2.shape
    Dout = Wf3.shape[1]
    return pl.pallas_call(
        _head_body,
        in_specs=[
            pl.BlockSpec((NUM_GRAPHS, Dh), lambda: (0, 0)),
            pl.BlockSpec((Dh, Dfc), lambda: (0, 0)),
            pl.BlockSpec((1, Dfc), lambda: (0, 0)),
            pl.BlockSpec((1, Dfc), lambda: (0, 0)),
            pl.BlockSpec((1, Dfc), lambda: (0, 0)),
            pl.BlockSpec((Dfc, Dout), lambda: (0, 0)),
            pl.BlockSpec((1, Dout), lambda: (0, 0)),
        ],
        out_specs=pl.BlockSpec((NUM_GRAPHS, Dout), lambda: (0, 0)),
        out_shape=jax.ShapeDtypeStruct((NUM_GRAPHS, Dout), jnp.float32),
    )(hg, Wf2, bf2.reshape(1, Dfc), g4.reshape(1, Dfc), be4.reshape(1, Dfc),
      Wf3, bf3.reshape(1, Dout))


def kernel(x, edge_index, batch, W1, b1, g1, be1, W2, b2, g2, be2,
           W3, b3, g3, be3, Wf2, bf2, g4, be4, Wf3, bf3):
    src = edge_index[0]
    dst = edge_index[1]
    npadding = EPAD - E
    src_pad = jnp.concatenate([src, jnp.zeros((npadding,), jnp.int32)])
    dst_pad = jnp.concatenate([dst, jnp.full((npadding,), N, jnp.int32)])
    srcA = jnp.stack([src_pad, src_pad])
    srcB = jnp.stack([src_pad, src_pad + N])
    zeros = jnp.zeros((NPAD, DH), jnp.float32)
    batch3d = batch.reshape(GRID, 1, R)

    p = _agg_split(x, srcA, dst_pad, zeros)
    y, s = _mmstats1(x, p, W1, b1)
    h = _bnrelu(y, s, g1, be1)

    a = _agg_cols(h.reshape(NC * N, DH), srcB, dst_pad, zeros)
    y, s = _mmstats23(h, a, W2, b2)
    h = _bnrelu(y, s, g2, be2)

    a = _agg_cols(h.reshape(NC * N, DH), srcB, dst_pad, zeros)
    y, s = _mmstats23(h, a, W3, b3)
    hg = _pool(y, s, g3, be3, batch3d)

    return _head(hg, Wf2, bf2, g4, be4, Wf3, bf3)


# speedup vs baseline: 4.2526x; 1.3501x over previous
import functools

import jax
import jax.numpy as jnp
from jax import lax
from jax.experimental import pallas as pl
from jax.experimental.pallas import tpu as pltpu
from jax.experimental.pallas import tpu_sc as plsc

N = 10000
E = 320000
NUM_GRAPHS = 64

NC = 2
NS = 16
K = 128
EPAD = 4096 * 79
NPAD = 10240
RPT = NPAD // NS
DH = 128

_MESH = plsc.VectorSubcoreMesh(core_axis_name="c", subcore_axis_name="s")


def _make_agg(split_cores, CK):
    TPT = EPAD // (NC * NS) if split_cores else EPAD // NS
    TRIPS = TPT // CK
    PAIRS = TRIPS // 2

    @functools.partial(
        pl.kernel,
        out_type=jax.ShapeDtypeStruct((NC, NPAD, DH), jnp.float32),
        mesh=_MESH,
        scratch_types=[
            pltpu.VMEM((CK,), jnp.int32),
            pltpu.VMEM((CK,), jnp.int32),
            pltpu.VMEM((CK,), jnp.int32),
            pltpu.VMEM((CK, DH), jnp.float32),
            pltpu.VMEM((CK, DH), jnp.float32),
            pltpu.VMEM_SHARED((NPAD, DH), jnp.float32),
            pltpu.SemaphoreType.DMA,
            pltpu.SemaphoreType.DMA,
        ],
    )
    def agg(h_hbm, src_hbm, dst_hbm, zero_hbm, out_hbm,
            idx_a, idx_b, dst_v, stage_a, stage_b, acc_sh, sem_a, sem_b):
        cidx = lax.axis_index("c")
        sidx = lax.axis_index("s")

        rbase = sidx * RPT
        pltpu.sync_copy(zero_hbm.at[pl.ds(rbase, RPT)],
                        acc_sh.at[pl.ds(rbase, RPT)])
        plsc.subcore_barrier()

        if split_cores:
            ebase = (cidx * NS + sidx) * TPT
        else:
            ebase = sidx * TPT

        def issue(g, idx_v, stage, sem):
            e0 = jnp.minimum(ebase + g * CK, EPAD - CK)
            pltpu.sync_copy(src_hbm.at[cidx, pl.ds(e0, CK)], idx_v)
            pltpu.async_copy(h_hbm.at[idx_v], stage, sem)

        def drain(stage, sem):
            pltpu.make_async_copy(h_hbm.at[pl.ds(0, CK)], stage, sem).wait()

        def scatter(g, stage):
            e0 = ebase + g * CK
            pltpu.sync_copy(dst_hbm.at[pl.ds(e0, CK)], dst_v)
            pltpu.sync_copy(stage, acc_sh.at[dst_v], add=True)

        issue(0, idx_a, stage_a, sem_a)
        issue(1, idx_b, stage_b, sem_b)

        def body(p, carry):
            g = 2 * p
            drain(stage_a, sem_a)
            scatter(g, stage_a)
            issue(g + 2, idx_a, stage_a, sem_a)
            drain(stage_b, sem_b)
            scatter(g + 1, stage_b)
            issue(g + 3, idx_b, stage_b, sem_b)
            return carry

        lax.fori_loop(0, PAIRS, body, 0)
        drain(stage_a, sem_a)
        if TRIPS % 2:
            scatter(TRIPS - 1, stage_a)
        drain(stage_b, sem_b)

        plsc.subcore_barrier()

        pltpu.sync_copy(acc_sh.at[pl.ds(rbase, RPT)],
                        out_hbm.at[cidx, pl.ds(rbase, RPT)])

    return agg


_agg_split = _make_agg(True, 128)
_agg_cols = _make_agg(False, 128)


R = 400
GRID = N // R


def _acc_stats(y, s_ref):
    @pl.when(pl.program_id(0) == 0)
    def _():
        s_ref[...] = jnp.zeros_like(s_ref)

    s_ref[0:1, :] += jnp.sum(y, axis=0, keepdims=True)
    s_ref[1:2, :] += jnp.sum(y * y, axis=0, keepdims=True)


_PREC = lax.Precision.HIGHEST


def _mm1_body(x_ref, a_ref, w_ref, b_ref, y_ref, s_ref):
    s = x_ref[...] + a_ref[0] + a_ref[1]
    y = jnp.dot(s, w_ref[...], preferred_element_type=jnp.float32) + b_ref[...]
    y_ref[...] = y
    _acc_stats(y, s_ref)


def _mmstats1(x, agg, W, b):
    Din, Dout = W.shape
    return pl.pallas_call(
        _mm1_body,
        grid=(GRID,),
        in_specs=[
            pl.BlockSpec((R, Din), lambda i: (i, 0)),
            pl.BlockSpec((NC, R, Din), lambda i: (0, i, 0)),
            pl.BlockSpec((Din, Dout), lambda i: (0, 0)),
            pl.BlockSpec((1, Dout), lambda i: (0, 0)),
        ],
        out_specs=[
            pl.BlockSpec((R, Dout), lambda i: (i, 0)),
            pl.BlockSpec((2, Dout), lambda i: (0, 0)),
        ],
        out_shape=[
            jax.ShapeDtypeStruct((N, Dout), jnp.float32),
            jax.ShapeDtypeStruct((2, Dout), jnp.float32),
        ],
    )(x, agg, W, b.reshape(1, Dout))


def _mm23_body(h_ref, a_ref, w_ref, b_ref, y_ref, s_ref):
    s0 = h_ref[0] + a_ref[0]
    s1 = h_ref[1] + a_ref[1]
    y = (jnp.dot(s0, w_ref[0:DH, :], preferred_element_type=jnp.float32)
         + jnp.dot(s1, w_ref[DH:2 * DH, :], preferred_element_type=jnp.float32)
         + b_ref[...])
    y_ref[...] = y
    _acc_stats(y, s_ref)


def _mmstats23(h, agg, W, b):
    Dout = W.shape[1]
    return pl.pallas_call(
        _mm23_body,
        grid=(GRID,),
        in_specs=[
            pl.BlockSpec((NC, R, DH), lambda i: (0, i, 0)),
            pl.BlockSpec((NC, R, DH), lambda i: (0, i, 0)),
            pl.BlockSpec((2 * DH, Dout), lambda i: (0, 0)),
            pl.BlockSpec((1, Dout), lambda i: (0, 0)),
        ],
        out_specs=[
            pl.BlockSpec((R, Dout), lambda i: (i, 0)),
            pl.BlockSpec((2, Dout), lambda i: (0, 0)),
        ],
        out_shape=[
            jax.ShapeDtypeStruct((N, Dout), jnp.float32),
            jax.ShapeDtypeStruct((2, Dout), jnp.float32),
        ],
    )(h, agg, W, b.reshape(1, Dout))


def _bn_from_stats(s_ref, y):
    m = s_ref[0:1, :] / N
    v = s_ref[1:2, :] / N - m * m
    r = lax.rsqrt(v + 1e-5)
    return (y - m) * r


def _bnrelu_body(y_ref, s_ref, g_ref, be_ref, h_ref):
    h = _bn_from_stats(s_ref, y_ref[...]) * g_ref[...] + be_ref[...]
    h = jnp.maximum(h, 0.0)
    h_ref[0] = h[:, 0:DH]
    h_ref[1] = h[:, DH:2 * DH]


def _bnrelu(y, s, g, be):
    Dout = y.shape[1]
    return pl.pallas_call(
        _bnrelu_body,
        grid=(GRID,),
        in_specs=[
            pl.BlockSpec((R, Dout), lambda i: (i, 0)),
            pl.BlockSpec((2, Dout), lambda i: (0, 0)),
            pl.BlockSpec((1, Dout), lambda i: (0, 0)),
            pl.BlockSpec((1, Dout), lambda i: (0, 0)),
        ],
        out_specs=pl.BlockSpec((NC, R, DH), lambda i: (0, i, 0)),
        out_shape=jax.ShapeDtypeStruct((NC, N, DH), jnp.float32),
    )(y, s, g.reshape(1, Dout), be.reshape(1, Dout))


def _pool_body(y_ref, s_ref, g_ref, be_ref, b_ref, hg_ref):
    h = _bn_from_stats(s_ref, y_ref[...]) * g_ref[...] + be_ref[...]
    h = jnp.maximum(h, 0.0)
    bb = b_ref[0]
    gids = lax.broadcasted_iota(jnp.int32, (NUM_GRAPHS, R), 0)
    oh = (bb == gids).astype(jnp.float32)

    @pl.when(pl.program_id(0) == 0)
    def _():
        hg_ref[...] = jnp.zeros_like(hg_ref)

    hg_ref[...] += jnp.dot(oh, h, precision=_PREC,
                           preferred_element_type=jnp.float32)


def _pool(y, s, g, be, batch3d):
    Dout = y.shape[1]
    return pl.pallas_call(
        _pool_body,
        grid=(GRID,),
        in_specs=[
            pl.BlockSpec((R, Dout), lambda i: (i, 0)),
            pl.BlockSpec((2, Dout), lambda i: (0, 0)),
            pl.BlockSpec((1, Dout), lambda i: (0, 0)),
            pl.BlockSpec((1, Dout), lambda i: (0, 0)),
            pl.BlockSpec((1, 1, R), lambda i: (i, 0, 0)),
        ],
        out_specs=pl.BlockSpec((NUM_GRAPHS, Dout), lambda i: (0, 0)),
        out_shape=jax.ShapeDtypeStruct((NUM_GRAPHS, Dout), jnp.float32),
    )(y, s, g.reshape(1, Dout), be.reshape(1, Dout), batch3d)


def _softplus(x):
    return jnp.maximum(x, 0.0) + jnp.log1p(jnp.exp(-jnp.abs(x)))


def _head_body(hg_ref, w2_ref, b2_ref, g4_ref, be4_ref, w3_ref, b3_ref, o_ref):
    sp = _softplus(hg_ref[...])
    z = jnp.dot(sp, w2_ref[...], preferred_element_type=jnp.float32) + b2_ref[...]
    m = jnp.mean(z, axis=0, keepdims=True)
    v = jnp.mean(z * z, axis=0, keepdims=True) - m * m
    z = (z - m) * lax.rsqrt(v + 1e-5) * g4_ref[...] + be4_ref[...]
    z = _softplus(z)
    o = jnp.dot(z, w3_ref[...], preferred_element_type=jnp.float32) + b3_ref[...]
    nrm = jnp.maximum(jnp.sqrt(jnp.sum(o * o, axis=1, keepdims=True)), 1e-12)
    o_ref[...] = o / nrm


def _head(hg, Wf2, bf2, g4, be4, Wf3, bf3):
    Dh, Dfc = Wf2.shape
    Dout = Wf3.shape[1]
    return pl.pallas_call(
        _head_body,
        in_specs=[
            pl.BlockSpec((NUM_GRAPHS, Dh), lambda: (0, 0)),
            pl.BlockSpec((Dh, Dfc), lambda: (0, 0)),
            pl.BlockSpec((1, Dfc), lambda: (0, 0)),
            pl.BlockSpec((1, Dfc), lambda: (0, 0)),
            pl.BlockSpec((1, Dfc), lambda: (0, 0)),
            pl.BlockSpec((Dfc, Dout), lambda: (0, 0)),
            pl.BlockSpec((1, Dout), lambda: (0, 0)),
        ],
        out_specs=pl.BlockSpec((NUM_GRAPHS, Dout), lambda: (0, 0)),
        out_shape=jax.ShapeDtypeStruct((NUM_GRAPHS, Dout), jnp.float32),
    )(hg, Wf2, bf2.reshape(1, Dfc), g4.reshape(1, Dfc), be4.reshape(1, Dfc),
      Wf3, bf3.reshape(1, Dout))


def kernel(x, edge_index, batch, W1, b1, g1, be1, W2, b2, g2, be2,
           W3, b3, g3, be3, Wf2, bf2, g4, be4, Wf3, bf3):
    src = edge_index[0]
    dst = edge_index[1]
    npadding = EPAD - E
    src_pad = jnp.concatenate([src, jnp.zeros((npadding,), jnp.int32)])
    dst_pad = jnp.concatenate([dst, jnp.full((npadding,), N, jnp.int32)])
    srcA = jnp.stack([src_pad, src_pad])
    srcB = jnp.stack([src_pad, src_pad + N])
    zeros = jnp.zeros((NPAD, DH), jnp.float32)
    batch3d = batch.reshape(GRID, 1, R)

    p = _agg_split(x, srcA, dst_pad, zeros)
    y, s = _mmstats1(x, p, W1, b1)
    h = _bnrelu(y, s, g1, be1)

    a = _agg_cols(h.reshape(NC * N, DH), srcB, dst_pad, zeros)
    y, s = _mmstats23(h, a, W2, b2)
    h = _bnrelu(y, s, g2, be2)

    a = _agg_cols(h.reshape(NC * N, DH), srcB, dst_pad, zeros)
    y, s = _mmstats23(h, a, W3, b3)
    hg = _pool(y, s, g3, be3, batch3d)

    return _head(hg, Wf2, bf2, g4, be4, Wf3, bf3)
